# baseline (speedup 1.0000x reference)
import functools

import jax
import jax.numpy as jnp
from jax import lax
from jax.experimental import pallas as pl
from jax.experimental.pallas import tpu as pltpu
from jax.experimental.pallas import tpu_sc as plsc

_NCHUNK = 4


def _gather_rows(x, idx):
    B = idx.shape[0]
    D = x.shape[1]
    NS = plsc.get_sparse_core_info().num_subcores
    b_per_w = B // NS
    ch = b_per_w // _NCHUNK
    mesh = plsc.VectorSubcoreMesh(
        core_axis_name="c", subcore_axis_name="s", num_cores=1
    )

    @functools.partial(
        pl.kernel,
        mesh=mesh,
        out_type=jax.ShapeDtypeStruct((B, D), x.dtype),
        scratch_types=[
            pltpu.VMEM((b_per_w,), jnp.int32),
            pltpu.VMEM((b_per_w, D), x.dtype),
        ]
        + [pltpu.SemaphoreType.DMA] * (2 * _NCHUNK),
    )
    def k(x_hbm, idx_hbm, out_hbm, idx_v, rows_v, *sems):
        isems, gsems = sems[:_NCHUNK], sems[_NCHUNK:]
        base = lax.axis_index("s") * b_per_w
        ic = [
            pltpu.async_copy(
                idx_hbm.at[pl.ds(base + c * ch, ch)],
                idx_v.at[pl.ds(c * ch, ch)],
                isems[c],
            )
            for c in range(_NCHUNK)
        ]
        gc = []
        for c in range(_NCHUNK):
            ic[c].wait()
            gc.append(
                pltpu.async_copy(
                    x_hbm.at[idx_v.at[pl.ds(c * ch, ch)]],
                    rows_v.at[pl.ds(c * ch, ch)],
                    gsems[c],
                )
            )
        wc = []
        for c in range(_NCHUNK):
            gc[c].wait()
            wc.append(
                pltpu.async_copy(
                    rows_v.at[pl.ds(c * ch, ch)],
                    out_hbm.at[pl.ds(base + c * ch, ch)],
                    isems[c],
                )
            )
        for c in range(_NCHUNK):
            wc[c].wait()

    return k(x, idx)


def kernel(x, component_starts):
    idx = component_starts.astype(jnp.int32)
    return _gather_rows(x, idx)


# speedup vs baseline: 1.0107x; 1.0107x over previous
import functools

import jax
import jax.numpy as jnp
from jax import lax
from jax.experimental import pallas as pl
from jax.experimental.pallas import tpu as pltpu
from jax.experimental.pallas import tpu_sc as plsc


def _gather_rows(x, idx):
    B = idx.shape[0]
    D = x.shape[1]
    info = plsc.get_sparse_core_info()
    NS = info.num_subcores
    NC = 1
    NW = NC * NS
    b_per_w = B // NW
    h = b_per_w // 2
    mesh = plsc.VectorSubcoreMesh(
        core_axis_name="c", subcore_axis_name="s", num_cores=NC
    )

    @functools.partial(
        pl.kernel,
        mesh=mesh,
        out_type=jax.ShapeDtypeStruct((B, D), x.dtype),
        scratch_types=[
            pltpu.VMEM((b_per_w,), jnp.int32),
            pltpu.VMEM((b_per_w, D), x.dtype),
            pltpu.SemaphoreType.DMA,
            pltpu.SemaphoreType.DMA,
            pltpu.SemaphoreType.DMA,
            pltpu.SemaphoreType.DMA,
        ],
    )
    def k(x_hbm, idx_hbm, out_hbm, idx_v, rows_v, isem0, isem1, gsem0, gsem1):
        wid = lax.axis_index("s") * NC + lax.axis_index("c")
        base = wid * b_per_w
        i0 = pltpu.async_copy(
            idx_hbm.at[pl.ds(base, h)], idx_v.at[pl.ds(0, h)], isem0
        )
        i1 = pltpu.async_copy(
            idx_hbm.at[pl.ds(base + h, h)], idx_v.at[pl.ds(h, h)], isem1
        )
        i0.wait()
        g0 = pltpu.async_copy(
            x_hbm.at[idx_v.at[pl.ds(0, h)]], rows_v.at[pl.ds(0, h)], gsem0
        )
        i1.wait()
        g1 = pltpu.async_copy(
            x_hbm.at[idx_v.at[pl.ds(h, h)]], rows_v.at[pl.ds(h, h)], gsem1
        )
        g0.wait()
        w0 = pltpu.async_copy(
            rows_v.at[pl.ds(0, h)], out_hbm.at[pl.ds(base, h)], isem0
        )
        g1.wait()
        w1 = pltpu.async_copy(
            rows_v.at[pl.ds(h, h)], out_hbm.at[pl.ds(base + h, h)], isem1
        )
        w0.wait()
        w1.wait()

    return k(x, idx)


def kernel(x, component_starts):
    idx = component_starts.astype(jnp.int32)
    return _gather_rows(x, idx)
